# Initial kernel scaffold; baseline (speedup 1.0000x reference)
import functools

import jax
import jax.numpy as jnp
from jax import lax
from jax.experimental import pallas as pl
from jax.experimental.pallas import tpu as pltpu
from jax.experimental.pallas import tpu_sc as plsc

_NC = 2
_NS = 16
_CHUNK = 128


def _make_sc_scatter(n_pad, feat, chunks_per_tile):
  mesh = plsc.VectorSubcoreMesh(core_axis_name="c", subcore_axis_name="s")
  rows_per_tile = n_pad // _NS

  @functools.partial(
      pl.kernel,
      out_type=jax.ShapeDtypeStruct((_NC, n_pad, feat), jnp.float32),
      mesh=mesh,
      scratch_types=[
          pltpu.VMEM((_CHUNK,), jnp.int32),
          pltpu.VMEM((_CHUNK,), jnp.int32),
          pltpu.VMEM((_CHUNK, feat), jnp.float32),
          pltpu.VMEM_SHARED((n_pad, feat), jnp.float32),
          pltpu.SemaphoreType.DMA,
      ],
  )
  def sc_scatter(p_hbm, src_hbm, dst_hbm, zeros_hbm, out_hbm,
                 src_v, dst_v, rows_v, agg_sh, sem):
    c = lax.axis_index("c")
    s = lax.axis_index("s")
    r0 = s * rows_per_tile
    pltpu.sync_copy(zeros_hbm.at[pl.ds(r0, rows_per_tile)],
                    agg_sh.at[pl.ds(r0, rows_per_tile)])
    plsc.subcore_barrier()

    edge_base = (c * _NS + s) * chunks_per_tile * _CHUNK

    def body(j, carry):
      off = edge_base + j * _CHUNK
      pltpu.sync_copy(src_hbm.at[pl.ds(off, _CHUNK)], src_v)
      pltpu.sync_copy(dst_hbm.at[pl.ds(off, _CHUNK)], dst_v)
      pltpu.async_copy(p_hbm.at[src_v], rows_v, sem).wait()
      pltpu.sync_copy(rows_v, agg_sh.at[dst_v], add=True)
      return carry

    lax.fori_loop(0, chunks_per_tile, body, 0)
    plsc.subcore_barrier()
    pltpu.sync_copy(agg_sh.at[pl.ds(r0, rows_per_tile)],
                    out_hbm.at[c, pl.ds(r0, rows_per_tile)])

  return sc_scatter


def _tc_project(x, w_nbr, w_root, b, block_rows):
  n, d = x.shape
  h = w_nbr.shape[1]

  def body(x_ref, wn_ref, wr_ref, b_ref, p_ref, r_ref):
    xb = x_ref[...]
    p_ref[...] = jnp.dot(xb, wn_ref[...], preferred_element_type=jnp.float32)
    r_ref[...] = (jnp.dot(xb, wr_ref[...], preferred_element_type=jnp.float32)
                  + b_ref[...])

  return pl.pallas_call(
      body,
      grid=(n // block_rows,),
      in_specs=[
          pl.BlockSpec((block_rows, d), lambda i: (i, 0)),
          pl.BlockSpec((d, h), lambda i: (0, 0)),
          pl.BlockSpec((d, h), lambda i: (0, 0)),
          pl.BlockSpec((1, h), lambda i: (0, 0)),
      ],
      out_specs=[
          pl.BlockSpec((block_rows, h), lambda i: (i, 0)),
          pl.BlockSpec((block_rows, h), lambda i: (i, 0)),
      ],
      out_shape=[jax.ShapeDtypeStruct((n, h), jnp.float32)] * 2,
  )(x, w_nbr, w_root, b.reshape(1, h))


def _tc_combine_project(a0, a1, r, alpha, w_nbr, w_root, b, block_rows):
  n, h = a0.shape
  h2 = w_nbr.shape[1]

  def body(a0_ref, a1_ref, r_ref, al_ref, wn_ref, wr_ref, b_ref,
           p_ref, r2_ref):
    v = a0_ref[...] + a1_ref[...] + r_ref[...]
    v = jnp.where(v >= 0, v, al_ref[...] * v)
    p_ref[...] = jnp.dot(v, wn_ref[...], preferred_element_type=jnp.float32)
    r2_ref[...] = (jnp.dot(v, wr_ref[...], preferred_element_type=jnp.float32)
                   + b_ref[...])

  return pl.pallas_call(
      body,
      grid=(n // block_rows,),
      in_specs=[
          pl.BlockSpec((block_rows, h), lambda i: (i, 0)),
          pl.BlockSpec((block_rows, h), lambda i: (i, 0)),
          pl.BlockSpec((block_rows, h), lambda i: (i, 0)),
          pl.BlockSpec((1, h), lambda i: (0, 0)),
          pl.BlockSpec((h, h2), lambda i: (0, 0)),
          pl.BlockSpec((h, h2), lambda i: (0, 0)),
          pl.BlockSpec((1, h2), lambda i: (0, 0)),
      ],
      out_specs=[
          pl.BlockSpec((block_rows, h2), lambda i: (i, 0)),
          pl.BlockSpec((block_rows, h2), lambda i: (i, 0)),
      ],
      out_shape=[jax.ShapeDtypeStruct((n, h2), jnp.float32)] * 2,
  )(a0, a1, r, jnp.broadcast_to(alpha.reshape(1, 1), (1, h)), w_nbr, w_root,
    b.reshape(1, h2))


def _tc_pool_head(a0, a1, r, alpha, batch3d, w_out, b_out, num_graphs,
                  block_rows):
  n, h = a0.shape
  out_dim = w_out.shape[1]
  nblk = n // block_rows

  def body(a0_ref, a1_ref, r_ref, al_ref, bt_ref, wo_ref, bo_ref,
           out_ref, accs_ref, accc_ref):
    i = pl.program_id(0)
    v = a0_ref[...] + a1_ref[...] + r_ref[...]
    v = jnp.where(v >= 0, v, al_ref[...] * v)
    bvals = bt_ref[0]
    gids = lax.broadcasted_iota(jnp.int32, (num_graphs, block_rows), 0)
    onehot = (gids == bvals).astype(jnp.float32)
    sums = jnp.dot(onehot, v, preferred_element_type=jnp.float32)
    cnts = jnp.sum(onehot, axis=1, keepdims=True)

    @pl.when(i == 0)
    def _():
      accs_ref[...] = jnp.zeros_like(accs_ref)
      accc_ref[...] = jnp.zeros_like(accc_ref)

    accs_ref[...] += sums
    accc_ref[...] += cnts

    @pl.when(i == nblk - 1)
    def _():
      pooled = accs_ref[...] / jnp.maximum(accc_ref[...], 1.0)
      out_ref[...] = (jnp.dot(pooled, wo_ref[...],
                              preferred_element_type=jnp.float32)
                      + bo_ref[...])

  return pl.pallas_call(
      body,
      grid=(nblk,),
      in_specs=[
          pl.BlockSpec((block_rows, h), lambda i: (i, 0)),
          pl.BlockSpec((block_rows, h), lambda i: (i, 0)),
          pl.BlockSpec((block_rows, h), lambda i: (i, 0)),
          pl.BlockSpec((1, h), lambda i: (0, 0)),
          pl.BlockSpec((1, 1, block_rows), lambda i: (i, 0, 0)),
          pl.BlockSpec((h, out_dim), lambda i: (0, 0)),
          pl.BlockSpec((1, out_dim), lambda i: (0, 0)),
      ],
      out_specs=pl.BlockSpec((num_graphs, out_dim), lambda i: (0, 0)),
      out_shape=jax.ShapeDtypeStruct((num_graphs, out_dim), jnp.float32),
      scratch_shapes=[
          pltpu.VMEM((num_graphs, h), jnp.float32),
          pltpu.VMEM((num_graphs, 1), jnp.float32),
      ],
  )(a0, a1, r, jnp.broadcast_to(alpha.reshape(1, 1), (1, h)), batch3d,
    w_out, b_out.reshape(1, out_dim))


def kernel(x, edge_index, batch, W1_nbr, W1_root, b1, a1, W2_nbr, W2_root,
           b2, a2, W_out, b_out):
  n, d = x.shape
  hid = W1_nbr.shape[1]
  e = edge_index.shape[1]
  num_graphs = 64
  block_rows = 2000

  chunks_per_tile = -(-e // (_CHUNK * _NC * _NS))
  e_pad = chunks_per_tile * _CHUNK * _NC * _NS
  src = edge_index[0].astype(jnp.int32)
  dst = edge_index[1].astype(jnp.int32)
  src_p = jnp.concatenate([src, jnp.zeros((e_pad - e,), jnp.int32)])
  dst_p = jnp.concatenate([dst, jnp.full((e_pad - e,), n, jnp.int32)])

  n_pad = -(-(n + 1) // (_NS * 8)) * (_NS * 8)
  zeros = jnp.zeros((n_pad, hid), jnp.float32)

  sc_scatter = _make_sc_scatter(n_pad, hid, chunks_per_tile)

  batch3d = batch.astype(jnp.int32).reshape(n // block_rows, 1, block_rows)

  p1, r1 = _tc_project(x, W1_nbr, W1_root, b1, block_rows)
  agg1 = sc_scatter(p1, src_p, dst_p, zeros)
  p2, r2 = _tc_combine_project(agg1[0, :n], agg1[1, :n], r1, a1,
                               W2_nbr, W2_root, b2, block_rows)
  agg2 = sc_scatter(p2, src_p, dst_p, zeros)
  out = _tc_pool_head(agg2[0, :n], agg2[1, :n], r2, a2, batch3d,
                      W_out, b_out, num_graphs, block_rows)
  return out


# speedup vs baseline: 6.4138x; 6.4138x over previous
import functools

import jax
import jax.numpy as jnp
from jax import lax
from jax.experimental import pallas as pl
from jax.experimental.pallas import tpu as pltpu
from jax.experimental.pallas import tpu_sc as plsc

_NC = 2
_NS = 16
_CHUNK = 128


def _make_sc_scatter(n_pad, feat, chunks_per_tile):
  mesh = plsc.VectorSubcoreMesh(core_axis_name="c", subcore_axis_name="s")
  rows_per_tile = n_pad // _NS

  @functools.partial(
      pl.kernel,
      out_type=jax.ShapeDtypeStruct((_NC, n_pad, feat), jnp.float32),
      mesh=mesh,
      compiler_params=pltpu.CompilerParams(use_tc_tiling_on_sc=False),
      scratch_types=[
          pltpu.VMEM((_CHUNK,), jnp.int32),
          pltpu.VMEM((_CHUNK,), jnp.int32),
          pltpu.VMEM((_CHUNK, feat), jnp.float32),
          pltpu.VMEM_SHARED((n_pad, feat), jnp.float32),
          pltpu.SemaphoreType.DMA,
      ],
  )
  def sc_scatter(p_hbm, src_hbm, dst_hbm, zeros_hbm, out_hbm,
                 src_v, dst_v, rows_v, agg_sh, sem):
    c = lax.axis_index("c")
    s = lax.axis_index("s")
    r0 = s * rows_per_tile
    pltpu.sync_copy(zeros_hbm.at[pl.ds(r0, rows_per_tile)],
                    agg_sh.at[pl.ds(r0, rows_per_tile)])
    plsc.subcore_barrier()

    edge_base = (c * _NS + s) * chunks_per_tile * _CHUNK

    def body(j, carry):
      off = edge_base + j * _CHUNK
      pltpu.sync_copy(src_hbm.at[pl.ds(off, _CHUNK)], src_v)
      pltpu.sync_copy(dst_hbm.at[pl.ds(off, _CHUNK)], dst_v)
      pltpu.async_copy(p_hbm.at[src_v], rows_v, sem).wait()
      pltpu.sync_copy(rows_v, agg_sh.at[dst_v], add=True)
      return carry

    lax.fori_loop(0, chunks_per_tile, body, 0)
    plsc.subcore_barrier()
    pltpu.sync_copy(agg_sh.at[pl.ds(r0, rows_per_tile)],
                    out_hbm.at[c, pl.ds(r0, rows_per_tile)])

  return sc_scatter


def _tc_project(x, w_nbr, w_root, b, block_rows):
  n, d = x.shape
  h = w_nbr.shape[1]

  def body(x_ref, wn_ref, wr_ref, b_ref, p_ref, r_ref):
    xb = x_ref[...]
    p_ref[...] = jnp.dot(xb, wn_ref[...], preferred_element_type=jnp.float32)
    r_ref[...] = (jnp.dot(xb, wr_ref[...], preferred_element_type=jnp.float32)
                  + b_ref[...])

  return pl.pallas_call(
      body,
      grid=(n // block_rows,),
      in_specs=[
          pl.BlockSpec((block_rows, d), lambda i: (i, 0)),
          pl.BlockSpec((d, h), lambda i: (0, 0)),
          pl.BlockSpec((d, h), lambda i: (0, 0)),
          pl.BlockSpec((1, h), lambda i: (0, 0)),
      ],
      out_specs=[
          pl.BlockSpec((block_rows, h), lambda i: (i, 0)),
          pl.BlockSpec((block_rows, h), lambda i: (i, 0)),
      ],
      out_shape=[jax.ShapeDtypeStruct((n, h), jnp.float32)] * 2,
  )(x, w_nbr, w_root, b.reshape(1, h))


def _tc_combine_project(a0, a1, r, alpha, w_nbr, w_root, b, block_rows):
  n, h = a0.shape
  h2 = w_nbr.shape[1]

  def body(a0_ref, a1_ref, r_ref, al_ref, wn_ref, wr_ref, b_ref,
           p_ref, r2_ref):
    v = a0_ref[...] + a1_ref[...] + r_ref[...]
    v = jnp.where(v >= 0, v, al_ref[...] * v)
    p_ref[...] = jnp.dot(v, wn_ref[...], preferred_element_type=jnp.float32)
    r2_ref[...] = (jnp.dot(v, wr_ref[...], preferred_element_type=jnp.float32)
                   + b_ref[...])

  return pl.pallas_call(
      body,
      grid=(n // block_rows,),
      in_specs=[
          pl.BlockSpec((block_rows, h), lambda i: (i, 0)),
          pl.BlockSpec((block_rows, h), lambda i: (i, 0)),
          pl.BlockSpec((block_rows, h), lambda i: (i, 0)),
          pl.BlockSpec((1, h), lambda i: (0, 0)),
          pl.BlockSpec((h, h2), lambda i: (0, 0)),
          pl.BlockSpec((h, h2), lambda i: (0, 0)),
          pl.BlockSpec((1, h2), lambda i: (0, 0)),
      ],
      out_specs=[
          pl.BlockSpec((block_rows, h2), lambda i: (i, 0)),
          pl.BlockSpec((block_rows, h2), lambda i: (i, 0)),
      ],
      out_shape=[jax.ShapeDtypeStruct((n, h2), jnp.float32)] * 2,
  )(a0, a1, r, jnp.broadcast_to(alpha.reshape(1, 1), (1, h)), w_nbr, w_root,
    b.reshape(1, h2))


def _tc_pool_head(a0, a1, r, alpha, batch3d, w_out, b_out, num_graphs,
                  block_rows):
  n, h = a0.shape
  out_dim = w_out.shape[1]
  nblk = n // block_rows

  def body(a0_ref, a1_ref, r_ref, al_ref, bt_ref, wo_ref, bo_ref,
           out_ref, accs_ref, accc_ref):
    i = pl.program_id(0)
    v = a0_ref[...] + a1_ref[...] + r_ref[...]
    v = jnp.where(v >= 0, v, al_ref[...] * v)
    bvals = bt_ref[0]
    gids = lax.broadcasted_iota(jnp.int32, (num_graphs, block_rows), 0)
    onehot = (gids == bvals).astype(jnp.float32)
    sums = jnp.dot(onehot, v, preferred_element_type=jnp.float32)
    cnts = jnp.sum(onehot, axis=1, keepdims=True)

    @pl.when(i == 0)
    def _():
      accs_ref[...] = jnp.zeros_like(accs_ref)
      accc_ref[...] = jnp.zeros_like(accc_ref)

    accs_ref[...] += sums
    accc_ref[...] += cnts

    @pl.when(i == nblk - 1)
    def _():
      pooled = accs_ref[...] / jnp.maximum(accc_ref[...], 1.0)
      out_ref[...] = (jnp.dot(pooled, wo_ref[...],
                              preferred_element_type=jnp.float32)
                      + bo_ref[...])

  return pl.pallas_call(
      body,
      grid=(nblk,),
      in_specs=[
          pl.BlockSpec((block_rows, h), lambda i: (i, 0)),
          pl.BlockSpec((block_rows, h), lambda i: (i, 0)),
          pl.BlockSpec((block_rows, h), lambda i: (i, 0)),
          pl.BlockSpec((1, h), lambda i: (0, 0)),
          pl.BlockSpec((1, 1, block_rows), lambda i: (i, 0, 0)),
          pl.BlockSpec((h, out_dim), lambda i: (0, 0)),
          pl.BlockSpec((1, out_dim), lambda i: (0, 0)),
      ],
      out_specs=pl.BlockSpec((num_graphs, out_dim), lambda i: (0, 0)),
      out_shape=jax.ShapeDtypeStruct((num_graphs, out_dim), jnp.float32),
      scratch_shapes=[
          pltpu.VMEM((num_graphs, h), jnp.float32),
          pltpu.VMEM((num_graphs, 1), jnp.float32),
      ],
  )(a0, a1, r, jnp.broadcast_to(alpha.reshape(1, 1), (1, h)), batch3d,
    w_out, b_out.reshape(1, out_dim))


def kernel(x, edge_index, batch, W1_nbr, W1_root, b1, a1, W2_nbr, W2_root,
           b2, a2, W_out, b_out):
  n, d = x.shape
  hid = W1_nbr.shape[1]
  e = edge_index.shape[1]
  num_graphs = 64
  block_rows = 2000

  chunks_per_tile = -(-e // (_CHUNK * _NC * _NS))
  e_pad = chunks_per_tile * _CHUNK * _NC * _NS
  src = edge_index[0].astype(jnp.int32)
  dst = edge_index[1].astype(jnp.int32)
  src_p = jnp.concatenate([src, jnp.zeros((e_pad - e,), jnp.int32)])
  dst_p = jnp.concatenate([dst, jnp.full((e_pad - e,), n, jnp.int32)])

  n_pad = -(-(n + 1) // (_NS * 8)) * (_NS * 8)
  zeros = jnp.zeros((n_pad, hid), jnp.float32)

  sc_scatter = _make_sc_scatter(n_pad, hid, chunks_per_tile)

  batch3d = batch.astype(jnp.int32).reshape(n // block_rows, 1, block_rows)

  p1, r1 = _tc_project(x, W1_nbr, W1_root, b1, block_rows)
  agg1 = sc_scatter(p1, src_p, dst_p, zeros)
  p2, r2 = _tc_combine_project(agg1[0, :n], agg1[1, :n], r1, a1,
                               W2_nbr, W2_root, b2, block_rows)
  agg2 = sc_scatter(p2, src_p, dst_p, zeros)
  out = _tc_pool_head(agg2[0, :n], agg2[1, :n], r2, a2, batch3d,
                      W_out, b_out, num_graphs, block_rows)
  return out


# speedup vs baseline: 8.0903x; 1.2614x over previous
import functools

import jax
import jax.numpy as jnp
from jax import lax
from jax.experimental import pallas as pl
from jax.experimental.pallas import tpu as pltpu
from jax.experimental.pallas import tpu_sc as plsc

_NC = 2
_NS = 16
_CHUNK = 128


_NBUF = 4


def _make_sc_scatter(n_pad, feat, chunks_per_tile):
  mesh = plsc.VectorSubcoreMesh(core_axis_name="c", subcore_axis_name="s")
  rows_per_tile = n_pad // _NS
  assert chunks_per_tile % _NBUF == 0

  @functools.partial(
      pl.kernel,
      out_type=jax.ShapeDtypeStruct((_NC, n_pad, feat), jnp.float32),
      mesh=mesh,
      compiler_params=pltpu.CompilerParams(use_tc_tiling_on_sc=False),
      scratch_types=(
          [pltpu.VMEM((chunks_per_tile, _CHUNK), jnp.int32)] * 2
          + [pltpu.VMEM((_CHUNK, feat), jnp.float32)] * _NBUF
          + [pltpu.VMEM_SHARED((n_pad, feat), jnp.float32)]
          + [pltpu.SemaphoreType.DMA] * (3 + 2 * _NBUF)
      ),
  )
  def sc_scatter(p_hbm, src_hbm, dst_hbm, zeros_hbm, out_hbm, *scr):
    src_vm, dst_vm = scr[0], scr[1]
    rows = scr[2:2 + _NBUF]
    agg_sh = scr[2 + _NBUF]
    zsem, isem_s, isem_d = scr[3 + _NBUF:6 + _NBUF]
    gsem = scr[6 + _NBUF:6 + 2 * _NBUF]
    ssem = scr[6 + 2 * _NBUF:6 + 3 * _NBUF]

    c = lax.axis_index("c")
    s = lax.axis_index("s")
    r0 = s * rows_per_tile
    chunk0 = (c * _NS + s) * chunks_per_tile

    zcp = pltpu.async_copy(zeros_hbm.at[pl.ds(r0, rows_per_tile)],
                           agg_sh.at[pl.ds(r0, rows_per_tile)], zsem)
    icp_s = pltpu.async_copy(src_hbm.at[pl.ds(chunk0, chunks_per_tile)],
                             src_vm, isem_s)
    icp_d = pltpu.async_copy(dst_hbm.at[pl.ds(chunk0, chunks_per_tile)],
                             dst_vm, isem_d)
    icp_s.wait()
    icp_d.wait()
    for b in range(_NBUF):
      pltpu.async_copy(p_hbm.at[src_vm.at[b]], rows[b], gsem[b])
    zcp.wait()
    plsc.subcore_barrier()

    nsteps = chunks_per_tile // _NBUF

    def outer(jo, carry):
      j0 = jo * _NBUF
      for b in range(_NBUF):
        j = j0 + b
        pltpu.make_async_copy(p_hbm.at[src_vm.at[j]], rows[b], gsem[b]).wait()
        pltpu.async_copy(rows[b], agg_sh.at[dst_vm.at[j]], ssem[b], add=True)
        jn = j + _NBUF

        @pl.when(jn < chunks_per_tile)
        def _():
          pltpu.make_async_copy(rows[b], agg_sh.at[dst_vm.at[j]],
                                ssem[b]).wait()
          pltpu.async_copy(p_hbm.at[src_vm.at[jn]], rows[b], gsem[b])

      return carry

    lax.fori_loop(0, nsteps, outer, 0)
    for b in range(_NBUF):
      j = chunks_per_tile - _NBUF + b
      pltpu.make_async_copy(rows[b], agg_sh.at[dst_vm.at[j]], ssem[b]).wait()
    plsc.subcore_barrier()
    pltpu.sync_copy(agg_sh.at[pl.ds(r0, rows_per_tile)],
                    out_hbm.at[c, pl.ds(r0, rows_per_tile)])

  return sc_scatter


def _tc_project(x, w_nbr, w_root, b, block_rows):
  n, d = x.shape
  h = w_nbr.shape[1]

  def body(x_ref, wn_ref, wr_ref, b_ref, p_ref, r_ref):
    xb = x_ref[...]
    p_ref[...] = jnp.dot(xb, wn_ref[...], preferred_element_type=jnp.float32)
    r_ref[...] = (jnp.dot(xb, wr_ref[...], preferred_element_type=jnp.float32)
                  + b_ref[...])

  return pl.pallas_call(
      body,
      grid=(n // block_rows,),
      in_specs=[
          pl.BlockSpec((block_rows, d), lambda i: (i, 0)),
          pl.BlockSpec((d, h), lambda i: (0, 0)),
          pl.BlockSpec((d, h), lambda i: (0, 0)),
          pl.BlockSpec((1, h), lambda i: (0, 0)),
      ],
      out_specs=[
          pl.BlockSpec((block_rows, h), lambda i: (i, 0)),
          pl.BlockSpec((block_rows, h), lambda i: (i, 0)),
      ],
      out_shape=[jax.ShapeDtypeStruct((n, h), jnp.float32)] * 2,
  )(x, w_nbr, w_root, b.reshape(1, h))


def _tc_combine_project(a0, a1, r, alpha, w_nbr, w_root, b, block_rows):
  n, h = a0.shape
  h2 = w_nbr.shape[1]

  def body(a0_ref, a1_ref, r_ref, al_ref, wn_ref, wr_ref, b_ref,
           p_ref, r2_ref):
    v = a0_ref[...] + a1_ref[...] + r_ref[...]
    v = jnp.where(v >= 0, v, al_ref[...] * v)
    p_ref[...] = jnp.dot(v, wn_ref[...], preferred_element_type=jnp.float32)
    r2_ref[...] = (jnp.dot(v, wr_ref[...], preferred_element_type=jnp.float32)
                   + b_ref[...])

  return pl.pallas_call(
      body,
      grid=(n // block_rows,),
      in_specs=[
          pl.BlockSpec((block_rows, h), lambda i: (i, 0)),
          pl.BlockSpec((block_rows, h), lambda i: (i, 0)),
          pl.BlockSpec((block_rows, h), lambda i: (i, 0)),
          pl.BlockSpec((1, h), lambda i: (0, 0)),
          pl.BlockSpec((h, h2), lambda i: (0, 0)),
          pl.BlockSpec((h, h2), lambda i: (0, 0)),
          pl.BlockSpec((1, h2), lambda i: (0, 0)),
      ],
      out_specs=[
          pl.BlockSpec((block_rows, h2), lambda i: (i, 0)),
          pl.BlockSpec((block_rows, h2), lambda i: (i, 0)),
      ],
      out_shape=[jax.ShapeDtypeStruct((n, h2), jnp.float32)] * 2,
  )(a0, a1, r, jnp.broadcast_to(alpha.reshape(1, 1), (1, h)), w_nbr, w_root,
    b.reshape(1, h2))


def _tc_pool_head(a0, a1, r, alpha, batch3d, w_out, b_out, num_graphs,
                  block_rows):
  n, h = a0.shape
  out_dim = w_out.shape[1]
  nblk = n // block_rows

  def body(a0_ref, a1_ref, r_ref, al_ref, bt_ref, wo_ref, bo_ref,
           out_ref, accs_ref, accc_ref):
    i = pl.program_id(0)
    v = a0_ref[...] + a1_ref[...] + r_ref[...]
    v = jnp.where(v >= 0, v, al_ref[...] * v)
    bvals = bt_ref[0]
    gids = lax.broadcasted_iota(jnp.int32, (num_graphs, block_rows), 0)
    onehot = (gids == bvals).astype(jnp.float32)
    sums = jnp.dot(onehot, v, preferred_element_type=jnp.float32)
    cnts = jnp.sum(onehot, axis=1, keepdims=True)

    @pl.when(i == 0)
    def _():
      accs_ref[...] = jnp.zeros_like(accs_ref)
      accc_ref[...] = jnp.zeros_like(accc_ref)

    accs_ref[...] += sums
    accc_ref[...] += cnts

    @pl.when(i == nblk - 1)
    def _():
      pooled = accs_ref[...] / jnp.maximum(accc_ref[...], 1.0)
      out_ref[...] = (jnp.dot(pooled, wo_ref[...],
                              preferred_element_type=jnp.float32)
                      + bo_ref[...])

  return pl.pallas_call(
      body,
      grid=(nblk,),
      in_specs=[
          pl.BlockSpec((block_rows, h), lambda i: (i, 0)),
          pl.BlockSpec((block_rows, h), lambda i: (i, 0)),
          pl.BlockSpec((block_rows, h), lambda i: (i, 0)),
          pl.BlockSpec((1, h), lambda i: (0, 0)),
          pl.BlockSpec((1, 1, block_rows), lambda i: (i, 0, 0)),
          pl.BlockSpec((h, out_dim), lambda i: (0, 0)),
          pl.BlockSpec((1, out_dim), lambda i: (0, 0)),
      ],
      out_specs=pl.BlockSpec((num_graphs, out_dim), lambda i: (0, 0)),
      out_shape=jax.ShapeDtypeStruct((num_graphs, out_dim), jnp.float32),
      scratch_shapes=[
          pltpu.VMEM((num_graphs, h), jnp.float32),
          pltpu.VMEM((num_graphs, 1), jnp.float32),
      ],
  )(a0, a1, r, jnp.broadcast_to(alpha.reshape(1, 1), (1, h)), batch3d,
    w_out, b_out.reshape(1, out_dim))


def kernel(x, edge_index, batch, W1_nbr, W1_root, b1, a1, W2_nbr, W2_root,
           b2, a2, W_out, b_out):
  n, d = x.shape
  hid = W1_nbr.shape[1]
  e = edge_index.shape[1]
  num_graphs = 64
  block_rows = 2000

  chunks_per_tile = -(-e // (_CHUNK * _NC * _NS))
  chunks_per_tile = -(-chunks_per_tile // _NBUF) * _NBUF
  e_pad = chunks_per_tile * _CHUNK * _NC * _NS
  src = edge_index[0].astype(jnp.int32)
  dst = edge_index[1].astype(jnp.int32)
  src_p = jnp.concatenate([src, jnp.zeros((e_pad - e,), jnp.int32)])
  dst_p = jnp.concatenate([dst, jnp.full((e_pad - e,), n, jnp.int32)])
  src_p = src_p.reshape(e_pad // _CHUNK, _CHUNK)
  dst_p = dst_p.reshape(e_pad // _CHUNK, _CHUNK)

  n_pad = -(-(n + 1) // (_NS * 8)) * (_NS * 8)
  zeros = jnp.zeros((n_pad, hid), jnp.float32)

  sc_scatter = _make_sc_scatter(n_pad, hid, chunks_per_tile)

  batch3d = batch.astype(jnp.int32).reshape(n // block_rows, 1, block_rows)

  p1, r1 = _tc_project(x, W1_nbr, W1_root, b1, block_rows)
  agg1 = sc_scatter(p1, src_p, dst_p, zeros)
  p2, r2 = _tc_combine_project(agg1[0, :n], agg1[1, :n], r1, a1,
                               W2_nbr, W2_root, b2, block_rows)
  agg2 = sc_scatter(p2, src_p, dst_p, zeros)
  out = _tc_pool_head(agg2[0, :n], agg2[1, :n], r2, a2, batch3d,
                      W_out, b_out, num_graphs, block_rows)
  return out


# speedup vs baseline: 8.8453x; 1.0933x over previous
import functools

import jax
import jax.numpy as jnp
from jax import lax
from jax.experimental import pallas as pl
from jax.experimental.pallas import tpu as pltpu
from jax.experimental.pallas import tpu_sc as plsc

_NC = 2
_NS = 16
_CHUNK = 128
_NBUF = 8
_C0_SHARE_NUM, _C0_SHARE_DEN = 4, 5


def _make_sc_scatter(n_pad, feat, total_chunks):
  mesh = plsc.VectorSubcoreMesh(core_axis_name="c", subcore_axis_name="s")
  rows_per_tile = n_pad // _NS
  t0 = (total_chunks * _C0_SHARE_NUM // (_C0_SHARE_DEN * _NS * _NBUF)) * (
      _NS * _NBUF)
  m0 = t0 // _NS
  m1 = (total_chunks - t0) // _NS
  assert m1 % _NBUF == 0 and m1 > 0

  @functools.partial(
      pl.kernel,
      out_type=jax.ShapeDtypeStruct((_NC, n_pad, feat), jnp.float32),
      mesh=mesh,
      compiler_params=pltpu.CompilerParams(use_tc_tiling_on_sc=False),
      scratch_types=(
          [pltpu.VMEM((max(m0, m1), _CHUNK), jnp.int32)] * 2
          + [pltpu.VMEM((_CHUNK, feat), jnp.float32)] * _NBUF
          + [pltpu.VMEM_SHARED((n_pad, feat), jnp.float32)]
          + [pltpu.SemaphoreType.DMA] * (3 + 2 * _NBUF)
      ),
  )
  def sc_scatter(p_hbm, edges_hbm, zeros_hbm, out_hbm, *scr):
    src_vm, dst_vm = scr[0], scr[1]
    rows = scr[2:2 + _NBUF]
    agg_sh = scr[2 + _NBUF]
    zsem, isem_s, isem_d = scr[3 + _NBUF:6 + _NBUF]
    gsem = scr[6 + _NBUF:6 + 2 * _NBUF]
    ssem = scr[6 + 2 * _NBUF:6 + 3 * _NBUF]

    c = lax.axis_index("c")
    s = lax.axis_index("s")
    r0 = s * rows_per_tile
    m = jnp.where(c == 0, m0, m1)
    chunk0 = jnp.where(c == 0, s * m0, t0 + s * m1)

    zcp = pltpu.async_copy(zeros_hbm.at[pl.ds(r0, rows_per_tile)],
                           agg_sh.at[pl.ds(r0, rows_per_tile)], zsem)

    @pl.when(c == 0)
    def _():
      pltpu.async_copy(edges_hbm.at[0, pl.ds(chunk0, m0)],
                       src_vm.at[pl.ds(0, m0)], isem_s)
      pltpu.async_copy(edges_hbm.at[1, pl.ds(chunk0, m0)],
                       dst_vm.at[pl.ds(0, m0)], isem_d)
      pltpu.make_async_copy(edges_hbm.at[0, pl.ds(chunk0, m0)],
                            src_vm.at[pl.ds(0, m0)], isem_s).wait()
      pltpu.make_async_copy(edges_hbm.at[1, pl.ds(chunk0, m0)],
                            dst_vm.at[pl.ds(0, m0)], isem_d).wait()

    @pl.when(c != 0)
    def _():
      pltpu.async_copy(edges_hbm.at[0, pl.ds(chunk0, m1)],
                       src_vm.at[pl.ds(0, m1)], isem_s)
      pltpu.async_copy(edges_hbm.at[1, pl.ds(chunk0, m1)],
                       dst_vm.at[pl.ds(0, m1)], isem_d)
      pltpu.make_async_copy(edges_hbm.at[0, pl.ds(chunk0, m1)],
                            src_vm.at[pl.ds(0, m1)], isem_s).wait()
      pltpu.make_async_copy(edges_hbm.at[1, pl.ds(chunk0, m1)],
                            dst_vm.at[pl.ds(0, m1)], isem_d).wait()

    for b in range(_NBUF):
      pltpu.async_copy(p_hbm.at[src_vm.at[b]], rows[b], gsem[b])
    zcp.wait()
    plsc.subcore_barrier()

    def outer(jo, carry):
      j0 = jo * _NBUF
      for b in range(_NBUF):
        j = j0 + b
        pltpu.make_async_copy(p_hbm.at[src_vm.at[j]], rows[b], gsem[b]).wait()
        pltpu.async_copy(rows[b], agg_sh.at[dst_vm.at[j]], ssem[b], add=True)
        jn = j + _NBUF

        @pl.when(jn < m)
        def _():
          pltpu.make_async_copy(rows[b], agg_sh.at[dst_vm.at[j]],
                                ssem[b]).wait()
          pltpu.async_copy(p_hbm.at[src_vm.at[jn]], rows[b], gsem[b])

      return carry

    lax.fori_loop(0, m // _NBUF, outer, 0)
    for b in range(_NBUF):
      j = m - _NBUF + b
      pltpu.make_async_copy(rows[b], agg_sh.at[dst_vm.at[j]], ssem[b]).wait()
    plsc.subcore_barrier()
    pltpu.sync_copy(agg_sh.at[pl.ds(r0, rows_per_tile)],
                    out_hbm.at[c, pl.ds(r0, rows_per_tile)])

  return sc_scatter


def _tc_project(x, w_nbr, w_root, b, block_rows):
  n, d = x.shape
  h = w_nbr.shape[1]

  def body(x_ref, wn_ref, wr_ref, b_ref, p_ref, r_ref):
    xb = x_ref[...]
    p_ref[...] = jnp.dot(xb, wn_ref[...], preferred_element_type=jnp.float32)
    r_ref[...] = (jnp.dot(xb, wr_ref[...], preferred_element_type=jnp.float32)
                  + b_ref[...])

  return pl.pallas_call(
      body,
      grid=(n // block_rows,),
      in_specs=[
          pl.BlockSpec((block_rows, d), lambda i: (i, 0)),
          pl.BlockSpec((d, h), lambda i: (0, 0)),
          pl.BlockSpec((d, h), lambda i: (0, 0)),
          pl.BlockSpec((1, h), lambda i: (0, 0)),
      ],
      out_specs=[
          pl.BlockSpec((block_rows, h), lambda i: (i, 0)),
          pl.BlockSpec((block_rows, h), lambda i: (i, 0)),
      ],
      out_shape=[jax.ShapeDtypeStruct((n, h), jnp.float32)] * 2,
  )(x, w_nbr, w_root, b.reshape(1, h))


def _tc_combine_project(agg, r, alpha, w_nbr, w_root, b, block_rows):
  n, h = r.shape
  h2 = w_nbr.shape[1]

  def body(a0_ref, a1_ref, r_ref, al_ref, wn_ref, wr_ref, b_ref,
           p_ref, r2_ref):
    v = a0_ref[0] + a1_ref[0] + r_ref[...]
    v = jnp.where(v >= 0, v, al_ref[...] * v)
    p_ref[...] = jnp.dot(v, wn_ref[...], preferred_element_type=jnp.float32)
    r2_ref[...] = (jnp.dot(v, wr_ref[...], preferred_element_type=jnp.float32)
                   + b_ref[...])

  return pl.pallas_call(
      body,
      grid=(n // block_rows,),
      in_specs=[
          pl.BlockSpec((1, block_rows, h), lambda i: (0, i, 0)),
          pl.BlockSpec((1, block_rows, h), lambda i: (1, i, 0)),
          pl.BlockSpec((block_rows, h), lambda i: (i, 0)),
          pl.BlockSpec((1, h), lambda i: (0, 0)),
          pl.BlockSpec((h, h2), lambda i: (0, 0)),
          pl.BlockSpec((h, h2), lambda i: (0, 0)),
          pl.BlockSpec((1, h2), lambda i: (0, 0)),
      ],
      out_specs=[
          pl.BlockSpec((block_rows, h2), lambda i: (i, 0)),
          pl.BlockSpec((block_rows, h2), lambda i: (i, 0)),
      ],
      out_shape=[jax.ShapeDtypeStruct((n, h2), jnp.float32)] * 2,
  )(agg, agg, r, jnp.broadcast_to(alpha.reshape(1, 1), (1, h)), w_nbr, w_root,
    b.reshape(1, h2))


def _tc_pool_head(agg, r, alpha, batch3d, w_out, b_out, num_graphs,
                  block_rows):
  n, h = r.shape
  out_dim = w_out.shape[1]
  nblk = n // block_rows

  def body(a0_ref, a1_ref, r_ref, al_ref, bt_ref, wo_ref, bo_ref,
           out_ref, accs_ref, accc_ref):
    i = pl.program_id(0)
    v = a0_ref[0] + a1_ref[0] + r_ref[...]
    v = jnp.where(v >= 0, v, al_ref[...] * v)
    bvals = bt_ref[0]
    gids = lax.broadcasted_iota(jnp.int32, (num_graphs, block_rows), 0)
    onehot = (gids == bvals).astype(jnp.float32)
    sums = jnp.dot(onehot, v, preferred_element_type=jnp.float32)
    cnts = jnp.sum(onehot, axis=1, keepdims=True)

    @pl.when(i == 0)
    def _():
      accs_ref[...] = jnp.zeros_like(accs_ref)
      accc_ref[...] = jnp.zeros_like(accc_ref)

    accs_ref[...] += sums
    accc_ref[...] += cnts

    @pl.when(i == nblk - 1)
    def _():
      pooled = accs_ref[...] / jnp.maximum(accc_ref[...], 1.0)
      out_ref[...] = (jnp.dot(pooled, wo_ref[...],
                              preferred_element_type=jnp.float32)
                      + bo_ref[...])

  return pl.pallas_call(
      body,
      grid=(nblk,),
      in_specs=[
          pl.BlockSpec((1, block_rows, h), lambda i: (0, i, 0)),
          pl.BlockSpec((1, block_rows, h), lambda i: (1, i, 0)),
          pl.BlockSpec((block_rows, h), lambda i: (i, 0)),
          pl.BlockSpec((1, h), lambda i: (0, 0)),
          pl.BlockSpec((1, 1, block_rows), lambda i: (i, 0, 0)),
          pl.BlockSpec((h, out_dim), lambda i: (0, 0)),
          pl.BlockSpec((1, out_dim), lambda i: (0, 0)),
      ],
      out_specs=pl.BlockSpec((num_graphs, out_dim), lambda i: (0, 0)),
      out_shape=jax.ShapeDtypeStruct((num_graphs, out_dim), jnp.float32),
      scratch_shapes=[
          pltpu.VMEM((num_graphs, h), jnp.float32),
          pltpu.VMEM((num_graphs, 1), jnp.float32),
      ],
  )(agg, agg, r, jnp.broadcast_to(alpha.reshape(1, 1), (1, h)), batch3d,
    w_out, b_out.reshape(1, out_dim))


def kernel(x, edge_index, batch, W1_nbr, W1_root, b1, a1, W2_nbr, W2_root,
           b2, a2, W_out, b_out):
  n, d = x.shape
  hid = W1_nbr.shape[1]
  e = edge_index.shape[1]
  num_graphs = 64
  block_rows = 2000

  grain = _CHUNK * _NC * _NS * _NBUF
  e_pad = -(-e // grain) * grain
  total_chunks = e_pad // _CHUNK
  ei = edge_index.astype(jnp.int32)
  pad_vals = jnp.stack([jnp.zeros((e_pad - e,), jnp.int32),
                        jnp.full((e_pad - e,), n, jnp.int32)])
  edges = jnp.concatenate([ei, pad_vals], axis=1).reshape(
      2, total_chunks, _CHUNK)

  n_pad = -(-(n + 1) // (_NS * 8)) * (_NS * 8)
  zeros = jnp.zeros((n_pad, hid), jnp.float32)

  sc_scatter = _make_sc_scatter(n_pad, hid, total_chunks)

  batch3d = batch.astype(jnp.int32).reshape(n // block_rows, 1, block_rows)

  p1, r1 = _tc_project(x, W1_nbr, W1_root, b1, block_rows)
  agg1 = sc_scatter(p1, edges, zeros)
  p2, r2 = _tc_combine_project(agg1, r1, a1, W2_nbr, W2_root, b2, block_rows)
  agg2 = sc_scatter(p2, edges, zeros)
  out = _tc_pool_head(agg2, r2, a2, batch3d, W_out, b_out, num_graphs,
                      block_rows)
  return out
